# baseline (device time: 5342 ns/iter reference)
import jax
import jax.numpy as jnp
from jax import lax
from jax.experimental import pallas as pl
from jax.experimental.pallas import tpu as pltpu

N_DEV = 4


def kernel(x):
    m, n = x.shape

    def body(x_ref, out_ref, xv_ref, total_ref, recv_ref, in_sem,
             send_sems, recv_sems):
        my = lax.axis_index("i")

        barrier = pltpu.get_barrier_semaphore()
        pl.semaphore_signal(barrier, inc=1)
        pl.semaphore_wait(barrier, 1)

        dma_in = pltpu.make_async_copy(x_ref, xv_ref, in_sem)
        dma_in.start()
        dma_in.wait()

        p = xv_ref[:, :]
        h = m
        while h > 1:
            h //= 2
            p = p[:h, :] * p[h:, :]
        total_ref[:, :] = p

        for j in range(N_DEV):
            for k in range(j + 1, N_DEV):

                @pl.when(my == j)
                def _send(j=j, k=k):
                    pltpu.make_async_remote_copy(
                        src_ref=total_ref,
                        dst_ref=recv_ref.at[j],
                        send_sem=send_sems.at[k - j - 1],
                        recv_sem=recv_sems.at[j],
                        device_id=(k,),
                        device_id_type=pl.DeviceIdType.MESH,
                    ).start()

        acc = xv_ref[:, :]
        s = 1
        while s < m // 2:
            acc = acc * jnp.concatenate(
                [jnp.ones((s, n), acc.dtype), acc[: m - s, :]], axis=0
            )
            s *= 2

        for j in range(N_DEV - 1):

            @pl.when(j < my)
            def _recv(j=j):
                pltpu.make_async_remote_copy(
                    src_ref=total_ref,
                    dst_ref=recv_ref.at[j],
                    send_sem=send_sems.at[0],
                    recv_sem=recv_sems.at[j],
                    device_id=(j,),
                    device_id_type=pl.DeviceIdType.MESH,
                ).wait_recv()

        prefix = jnp.ones((1, n), acc.dtype)
        for j in range(N_DEV - 1):
            prefix = jnp.where(j < my, prefix * recv_ref[j, :, :], prefix)

        half = m // 2
        top = acc[:half, :] * prefix
        out_ref[:half, :] = top.astype(jnp.bfloat16)
        out_ref[half:, :] = (acc[half:, :] * top).astype(jnp.bfloat16)

        for j in range(N_DEV):
            for k in range(j + 1, N_DEV):

                @pl.when(my == j)
                def _drain(j=j, k=k):
                    pltpu.make_async_remote_copy(
                        src_ref=total_ref,
                        dst_ref=recv_ref.at[j],
                        send_sem=send_sems.at[k - j - 1],
                        recv_sem=recv_sems.at[j],
                        device_id=(k,),
                        device_id_type=pl.DeviceIdType.MESH,
                    ).wait_send()

    return pl.pallas_call(
        body,
        out_shape=jax.ShapeDtypeStruct((m, n), jnp.bfloat16),
        in_specs=[pl.BlockSpec(memory_space=pl.ANY)],
        out_specs=pl.BlockSpec(memory_space=pltpu.VMEM),
        scratch_shapes=[
            pltpu.VMEM((m, n), x.dtype),
            pltpu.VMEM((1, n), x.dtype),
            pltpu.VMEM((N_DEV - 1, 1, n), x.dtype),
            pltpu.SemaphoreType.DMA,
            pltpu.SemaphoreType.DMA((N_DEV - 1,)),
            pltpu.SemaphoreType.DMA((N_DEV - 1,)),
        ],
        compiler_params=pltpu.CompilerParams(collective_id=0),
    )(x)


# device time: 5157 ns/iter; 1.0359x vs baseline; 1.0359x over previous
import jax
import jax.numpy as jnp
from jax import lax
from jax.experimental import pallas as pl
from jax.experimental.pallas import tpu as pltpu

N_DEV = 4


def kernel(x):
    m, n = x.shape

    def body(x_ref, out_ref, total_ref, recv_ref, send_sems, recv_sems):
        my = lax.axis_index("i")

        barrier = pltpu.get_barrier_semaphore()
        pl.semaphore_signal(barrier, inc=1)
        pl.semaphore_wait(barrier, 1)

        p = x_ref[:, :]
        h = m
        while h > 1:
            h //= 4
            p = (p[:h, :] * p[h : 2 * h, :]) * (
                p[2 * h : 3 * h, :] * p[3 * h :, :]
            )
        total_ref[:, :] = p

        for j in range(N_DEV):
            for k in range(j + 1, N_DEV):

                @pl.when(my == j)
                def _send(j=j, k=k):
                    pltpu.make_async_remote_copy(
                        src_ref=total_ref,
                        dst_ref=recv_ref.at[j],
                        send_sem=send_sems.at[k - j - 1],
                        recv_sem=recv_sems.at[j],
                        device_id=(k,),
                        device_id_type=pl.DeviceIdType.MESH,
                    ).start()

        acc = x_ref[:, :]
        s = 1
        while s < m // 2:
            acc = acc * jnp.concatenate(
                [jnp.ones((s, n), acc.dtype), acc[: m - s, :]], axis=0
            )
            s *= 2

        for j in range(N_DEV - 1):

            @pl.when(j < my)
            def _recv(j=j):
                pltpu.make_async_remote_copy(
                    src_ref=total_ref,
                    dst_ref=recv_ref.at[j],
                    send_sem=send_sems.at[0],
                    recv_sem=recv_sems.at[j],
                    device_id=(j,),
                    device_id_type=pl.DeviceIdType.MESH,
                ).wait_recv()

        prefix = jnp.ones((1, n), acc.dtype)
        for j in range(N_DEV - 1):
            prefix = jnp.where(j < my, prefix * recv_ref[j, :, :], prefix)

        half = m // 2
        top = acc[:half, :] * prefix
        out_ref[:half, :] = top.astype(jnp.bfloat16)
        out_ref[half:, :] = (acc[half:, :] * top).astype(jnp.bfloat16)

        for j in range(N_DEV):
            for k in range(j + 1, N_DEV):

                @pl.when(my == j)
                def _drain(j=j, k=k):
                    pltpu.make_async_remote_copy(
                        src_ref=total_ref,
                        dst_ref=recv_ref.at[j],
                        send_sem=send_sems.at[k - j - 1],
                        recv_sem=recv_sems.at[j],
                        device_id=(k,),
                        device_id_type=pl.DeviceIdType.MESH,
                    ).wait_send()

    return pl.pallas_call(
        body,
        out_shape=jax.ShapeDtypeStruct((m, n), jnp.bfloat16),
        in_specs=[pl.BlockSpec(memory_space=pltpu.VMEM)],
        out_specs=pl.BlockSpec(memory_space=pltpu.VMEM),
        scratch_shapes=[
            pltpu.VMEM((1, n), x.dtype),
            pltpu.VMEM((N_DEV - 1, 1, n), x.dtype),
            pltpu.SemaphoreType.DMA((N_DEV - 1,)),
            pltpu.SemaphoreType.DMA((N_DEV - 1,)),
        ],
        compiler_params=pltpu.CompilerParams(collective_id=0),
    )(x)
